# baseline (speedup 1.0000x reference)
import functools

import jax
import jax.numpy as jnp
from jax import lax
from jax.experimental import pallas as pl
from jax.experimental.pallas import tpu as pltpu
from jax.experimental.pallas import tpu_sc as plsc

_B, _S, _D = 1024, 200, 64
_N = _B * _S
_NC, _NS = 2, 16
_NW = _NC * _NS
_BPW = _N // _NW
_CH = 128
_K = 5
_NCHUNK = _BPW // _CH
assert _NCHUNK % _K == 0 and _BPW % _CH == 0 and _N % _NW == 0

_mesh = plsc.VectorSubcoreMesh(core_axis_name="c", subcore_axis_name="s")


@functools.partial(
    pl.kernel,
    out_type=jax.ShapeDtypeStruct((_N, _D), jnp.float32),
    mesh=_mesh,
    compiler_params=pltpu.CompilerParams(use_tc_tiling_on_sc=False),
    scratch_types=[
        pltpu.VMEM((_NCHUNK, _CH), jnp.int32),
        pltpu.VMEM((_K, _CH, _D), jnp.float32),
        pltpu.SemaphoreType.DMA,
        pltpu.SemaphoreType.DMA,
    ],
)
def _sc_gather(idx_hbm, table_hbm, out_hbm, idx_v, rows_v, gsem, osem):
    wid = lax.axis_index("s") * _NC + lax.axis_index("c")
    base = wid * _BPW
    pltpu.sync_copy(idx_hbm.at[wid], idx_v)

    @pl.loop(0, _NCHUNK, step=_K)
    def _step(j):
        gathers = [
            pltpu.async_copy(table_hbm.at[idx_v.at[j + b]], rows_v.at[b], gsem)
            for b in range(_K)
        ]
        stores = []
        for b in range(_K):
            gathers[b].wait()
            stores.append(
                pltpu.async_copy(
                    rows_v.at[b],
                    out_hbm.at[pl.ds(base + (j + b) * _CH, _CH)],
                    osem,
                )
            )
        for st in stores:
            st.wait()


def kernel(inp_ids, table):
    idx = inp_ids.reshape(_NW, _NCHUNK, _CH)
    out = _sc_gather(idx, table)
    return out.reshape(_B, _S, _D)


# speedup vs baseline: 1.0541x; 1.0541x over previous
import functools

import jax
import jax.numpy as jnp
from jax import lax
from jax.experimental import pallas as pl
from jax.experimental.pallas import tpu as pltpu
from jax.experimental.pallas import tpu_sc as plsc

_B, _S, _D = 1024, 200, 64
_N = _B * _S
_NC, _NS = 2, 16
_NW = _NC * _NS
_BPW = _N // _NW
_CH = 128
_K = 5
_NCHUNK = _BPW // _CH
assert _NCHUNK % _K == 0 and _BPW % _CH == 0 and _N % _NW == 0

_mesh = plsc.VectorSubcoreMesh(core_axis_name="c", subcore_axis_name="s")


@functools.partial(
    pl.kernel,
    out_type=jax.ShapeDtypeStruct((_N, _D), jnp.float32),
    mesh=_mesh,
    compiler_params=pltpu.CompilerParams(use_tc_tiling_on_sc=False),
    scratch_types=[
        pltpu.VMEM((_NCHUNK, _CH), jnp.int32),
        pltpu.VMEM((_K, _CH, 2 * _D), jnp.float32),
        pltpu.SemaphoreType.DMA,
        pltpu.SemaphoreType.DMA,
    ],
)
def _sc_gather(idx_hbm, table_hbm, out_hbm, idx_v, rows_v, gsem, osem):
    wid = lax.axis_index("s") * _NC + lax.axis_index("c")
    base = wid * _BPW
    pltpu.sync_copy(idx_hbm.at[wid], idx_v)

    @pl.loop(0, _NCHUNK, step=_K)
    def _step(j):
        gathers = [
            pltpu.async_copy(table_hbm.at[idx_v.at[j + b]], rows_v.at[b], gsem)
            for b in range(_K)
        ]
        stores = []
        for b in range(_K):
            gathers[b].wait()
            stores.append(
                pltpu.async_copy(
                    rows_v.at[b].at[:, pl.ds(0, _D)],
                    out_hbm.at[pl.ds(base + (j + b) * _CH, _CH)],
                    osem,
                )
            )
        for st in stores:
            st.wait()


def kernel(inp_ids, table):
    idx = inp_ids.reshape(_NW, _NCHUNK, _CH)
    tpad = jnp.pad(table, ((0, 0), (0, _D)))
    out = _sc_gather(idx, tpad)
    return out.reshape(_B, _S, _D)
